# baseline (speedup 1.0000x reference)
import functools

import jax
import jax.numpy as jnp
from jax import lax
from jax.experimental import pallas as pl
from jax.experimental.pallas import tpu as pltpu
from jax.experimental.pallas import tpu_sc as plsc

N = 10000
NPAD = 10240
E = 320000
NC = 2
NS = 16
NW = NC * NS
E2 = 327680
EPW = E2 // NW
CH = 80
NCHUNK = EPW // CH
RPT = NPAD // NS
DEGW = 16
ZB = 80

_mesh = plsc.VectorSubcoreMesh(
    core_axis_name="c", subcore_axis_name="s", num_cores=NC, num_subcores=NS)


SCH = 4
CHS = CH * SCH
NSUP = NCHUNK // SCH


def _agg_body(D, with_deg, g_h, src_h, dst_h, mp_h, *rest):
    if with_deg:
        (agg_out, deg_out, srcA, dstA, mpA, srcB, dstB, mpB,
         rows0, rows1, aggacc, semG0, semG1, semS0, semS1, semIA, semIB,
         ones, zdeg, degacc) = rest
    else:
        (agg_out, srcA, dstA, mpA, srcB, dstB, mpB,
         rows0, rows1, aggacc, semG0, semG1, semS0, semS1,
         semIA, semIB) = rest
    c = lax.axis_index("c")
    s = lax.axis_index("s")
    wid = c * NS + s
    ebase0 = wid * EPW
    rows = (rows0, rows1)
    semG = (semG0, semG1)
    semS = (semS0, semS1)
    bankA = (srcA, dstA, mpA, semIA)
    bankB = (srcB, dstB, mpB, semIB)

    def zrow(r, _):
        for k in range(D // 16):
            rows0[r, pl.ds(k * 16, 16)] = jnp.zeros((16,), jnp.float32)
        return 0
    lax.fori_loop(0, CH, zrow, 0)
    for k in range(RPT // CH):
        pltpu.sync_copy(rows0, aggacc.at[pl.ds(s * RPT + k * CH, CH)])
    if with_deg:
        def zd(i, _):
            zdeg[pl.ds(i * 16, 16)] = jnp.zeros((16,), jnp.float32)
            return 0
        lax.fori_loop(0, RPT // 16, zd, 0)
        pltpu.sync_copy(zdeg, degacc.at[pl.ds(s * RPT, RPT)])

        def od(i, _):
            ones[pl.ds(i * 16, 16)] = jnp.ones((16,), jnp.float32)
            return 0
        lax.fori_loop(0, CH // 16, od, 0)
    plsc.subcore_barrier()

    def loadidx(t_sup, bank):
        eb = ebase0 + t_sup * CHS
        pltpu.async_copy(src_h.at[pl.ds(eb, CHS)], bank[0], bank[3])
        pltpu.async_copy(dst_h.at[pl.ds(eb, CHS)], bank[1], bank[3])
        pltpu.async_copy(mp_h.at[pl.ds(eb, CHS)], bank[2], bank[3])

    def waitidx(bank):
        pltpu.make_async_copy(src_h.at[pl.ds(0, CHS)], bank[0], bank[3]).wait()
        pltpu.make_async_copy(dst_h.at[pl.ds(0, CHS)], bank[1], bank[3]).wait()
        pltpu.make_async_copy(mp_h.at[pl.ds(0, CHS)], bank[2], bank[3]).wait()

    def gstart(src_idx, b):
        pltpu.async_copy(g_h.at[src_idx], rows[b], semG[b])

    def wait_scat(b, dummy_idx):
        pltpu.make_async_copy(rows[b], aggacc.at[dummy_idx], semS[b]).wait()
        if with_deg:
            pltpu.make_async_copy(ones, degacc.at[dummy_idx], semS[b]).wait()

    def do_super(cur, nxt, last=False):
        srcv, dstv, mpv = cur[:3]
        for j in range(SCH):
            b = j % 2
            ob = (j + 1) % 2
            pltpu.make_async_copy(
                g_h.at[srcv.at[pl.ds(j * CH, CH)]], rows[b], semG[b]).wait()
            rb = rows[b]
            moff = j * CH

            def erow(i, _):
                for u in range(8):
                    e = i * 8 + u
                    spl = plsc.load_gather(
                        mpv, [jnp.full((16,), moff + e, jnp.int32)])
                    for k in range(D // 16):
                        sl = pl.ds(k * 16, 16)
                        rb[e, sl] = rb[e, sl] * spl
                return 0
            lax.fori_loop(0, CH // 8, erow, 0)
            if j < SCH - 1:
                if j > 0:
                    wait_scat(ob, dstv.at[pl.ds(0, CH)])
                gstart(srcv.at[pl.ds((j + 1) * CH, CH)], ob)
            elif not last:
                waitidx(nxt)
                wait_scat(ob, dstv.at[pl.ds(0, CH)])
                gstart(nxt[0].at[pl.ds(0, CH)], ob)
            dsl = dstv.at[pl.ds(j * CH, CH)]
            pltpu.async_copy(rb, aggacc.at[dsl], semS[b], add=True)
            if with_deg:
                pltpu.async_copy(ones, degacc.at[dsl], semS[b], add=True)

    loadidx(0, bankA)
    waitidx(bankA)
    gstart(srcA.at[pl.ds(0, CH)], 0)

    def pairbody(p, _):
        loadidx(2 * p + 1, bankB)
        do_super(bankA, bankB)
        wait_scat(1, dstA.at[pl.ds(0, CH)])
        loadidx(2 * p + 2, bankA)
        do_super(bankB, bankA)
        wait_scat(1, dstB.at[pl.ds(0, CH)])
        return 0
    lax.fori_loop(0, (NSUP - 2) // 2, pairbody, 0)

    loadidx(NSUP - 1, bankB)
    do_super(bankA, bankB)
    wait_scat(1, dstA.at[pl.ds(0, CH)])
    do_super(bankB, bankA, last=True)
    wait_scat(0, dstB.at[pl.ds(0, CH)])
    wait_scat(1, dstB.at[pl.ds(0, CH)])
    plsc.subcore_barrier()

    for k in range(RPT // CH):
        start = s * RPT + k * CH
        pltpu.sync_copy(aggacc.at[pl.ds(start, CH)], rows0)
        pltpu.sync_copy(rows0, agg_out.at[pl.ds(c * NPAD + start, CH)])
    if with_deg:
        pltpu.sync_copy(degacc.at[pl.ds(s * RPT, RPT)], zdeg)
        pltpu.sync_copy(zdeg, deg_out.at[pl.ds(c * NPAD + s * RPT, RPT)])


def _make_agg(D, with_deg):
    out_type = [jax.ShapeDtypeStruct((NC * NPAD, D), jnp.float32)]
    if with_deg:
        out_type.append(jax.ShapeDtypeStruct((NC * NPAD,), jnp.float32))
    bank = [
        pltpu.VMEM((CHS,), jnp.int32),
        pltpu.VMEM((CHS,), jnp.int32),
        pltpu.VMEM((CHS,), jnp.float32),
    ]
    scratch = bank + bank + [
        pltpu.VMEM((CH, D), jnp.float32),
        pltpu.VMEM((CH, D), jnp.float32),
        pltpu.VMEM_SHARED((NPAD, D), jnp.float32),
    ] + [pltpu.SemaphoreType.DMA] * 6
    if with_deg:
        scratch += [
            pltpu.VMEM((CH,), jnp.float32),
            pltpu.VMEM((RPT,), jnp.float32),
            pltpu.VMEM_SHARED((NPAD,), jnp.float32),
        ]
    return pl.kernel(
        functools.partial(_agg_body, D, with_deg),
        out_type=tuple(out_type) if with_deg else out_type[0],
        mesh=_mesh,
        scratch_types=scratch,
        compiler_params=pltpu.CompilerParams(needs_layout_passes=False),
    )


_agg128d = _make_agg(128, True)
_agg128 = _make_agg(128, False)


def _maskmul_body(a, b, o):
    o[...] = a[...] * b[...]


def _maskmul(m1, m2):
    m1r = m1.reshape(E // 128, 128)
    m2r = m2.reshape(E // 128, 128)
    out = pl.pallas_call(
        _maskmul_body,
        grid=(1,),
        in_specs=[pl.BlockSpec((E // 128, 128), lambda i: (0, 0)),
                  pl.BlockSpec((E // 128, 128), lambda i: (0, 0))],
        out_specs=pl.BlockSpec((E // 128, 128), lambda i: (0, 0)),
        out_shape=jax.ShapeDtypeStruct((E // 128, 128), jnp.float32),
    )(m1r, m2r)
    return out.reshape(E)


def _matmul_body(x, w, o):
    o[...] = jnp.dot(x[...], w[...], preferred_element_type=jnp.float32)


def _matmul(x, w):
    m, k = x.shape
    n = w.shape[1]
    bm = 1024
    return pl.pallas_call(
        _matmul_body,
        grid=(m // bm,),
        in_specs=[pl.BlockSpec((bm, k), lambda i: (i, 0)),
                  pl.BlockSpec((k, n), lambda i: (0, 0))],
        out_specs=pl.BlockSpec((bm, n), lambda i: (i, 0)),
        out_shape=jax.ShapeDtypeStruct((m, n), jnp.float32),
    )(x, w)


def _fuse1_body(g0, agg, deg, eps, a, w, out):
    degv = deg[...]
    dsum = degv[0, :, 0:1] + degv[1, :, 0:1]
    inv = 1.0 / jnp.maximum(dsum, 1.0)
    aggv = agg[...]
    ag = (aggv[0] + aggv[1]) * inv
    pre = (1.0 + eps[0, 0]) * g0[...] + ag
    h0 = jnp.where(pre >= 0.0, pre, a[0, 0] * pre)
    out[...] = jnp.dot(h0, w[...], preferred_element_type=jnp.float32)


def _fuse1(g0, agg, deg, eps, a, wcat):
    bm = 1024
    return pl.pallas_call(
        _fuse1_body,
        grid=(NPAD // bm,),
        in_specs=[
            pl.BlockSpec((bm, 128), lambda i: (i, 0)),
            pl.BlockSpec((NC, bm, 128), lambda i: (0, i, 0)),
            pl.BlockSpec((NC, bm, 1), lambda i: (0, i, 0)),
            pl.BlockSpec((1, 1), lambda i: (0, 0), memory_space=pltpu.SMEM),
            pl.BlockSpec((1, 1), lambda i: (0, 0), memory_space=pltpu.SMEM),
            pl.BlockSpec((128, 128), lambda i: (0, 0)),
        ],
        out_specs=pl.BlockSpec((bm, 128), lambda i: (i, 0)),
        out_shape=jax.ShapeDtypeStruct((NPAD, 128), jnp.float32),
    )(g0, agg, deg, eps, a, wcat)


def _fuse2_body(g1p, agg, deg, eps, a, out):
    degv = deg[...]
    dsum = degv[0, :, 0:1] + degv[1, :, 0:1]
    inv = 1.0 / jnp.maximum(dsum, 1.0)
    y = g1p[...]
    aggv = agg[...]
    ag = (aggv[0] + aggv[1])[:, :64] * inv
    pre = (1.0 + eps[0, 0]) * y[:, :64] + ag
    h1 = jnp.where(pre >= 0.0, pre, a[0, 0] * pre)
    out[...] = (y[:, 64:] + h1) * 0.5


def _fuse2(g1p, agg, deg, eps, a):
    bm = 1024
    return pl.pallas_call(
        _fuse2_body,
        grid=(NPAD // bm,),
        in_specs=[
            pl.BlockSpec((bm, 128), lambda i: (i, 0)),
            pl.BlockSpec((NC, bm, 128), lambda i: (0, i, 0)),
            pl.BlockSpec((NC, bm, 1), lambda i: (0, i, 0)),
            pl.BlockSpec((1, 1), lambda i: (0, 0), memory_space=pltpu.SMEM),
            pl.BlockSpec((1, 1), lambda i: (0, 0), memory_space=pltpu.SMEM),
        ],
        out_specs=pl.BlockSpec((bm, 64), lambda i: (i, 0)),
        out_shape=jax.ShapeDtypeStruct((NPAD, 64), jnp.float32),
    )(g1p, agg, deg, eps, a)


def kernel(h, snorm_n, snorm_e, mask1, mask2, eps0, eps1, a0, a1,
           W0, W1, Wpred, edge_index):
    npadE = E2 - E
    src2 = jnp.concatenate([edge_index[0],
                            jnp.zeros((npadE,), jnp.int32)])
    dst2 = jnp.concatenate([edge_index[1],
                            jnp.full((npadE,), NPAD - 1, jnp.int32)])
    mp = jnp.concatenate([_maskmul(mask1.reshape(E), mask2.reshape(E)),
                          jnp.zeros((npadE,), jnp.float32)])
    hpad = jnp.pad(h, ((0, NPAD - N), (0, 0)))

    g0 = _matmul(hpad, W0)
    agg0f, degf = _agg128d(g0, src2, dst2, mp)
    agg0 = agg0f.reshape(NC, NPAD, 128)
    deg = degf.reshape(NC, NPAD, 1)

    wcat = jnp.concatenate([W1, Wpred], axis=1)
    g1p = _fuse1(g0, agg0, deg, eps0.reshape(1, 1), a0.reshape(1, 1), wcat)

    agg1f = _agg128(g1p, src2, dst2, mp)
    agg1 = agg1f.reshape(NC, NPAD, 128)

    score = _fuse2(g1p, agg1, deg, eps1.reshape(1, 1), a1.reshape(1, 1))
    return score[:N][None]


# speedup vs baseline: 1.2189x; 1.2189x over previous
import functools

import jax
import jax.numpy as jnp
from jax import lax
from jax.experimental import pallas as pl
from jax.experimental.pallas import tpu as pltpu
from jax.experimental.pallas import tpu_sc as plsc

N = 10000
NPAD = 10240
E = 320000
NC = 2
NS = 16
NW = NC * NS
E2 = 327680
EPW = E2 // NW
CH = 80
NCHUNK = EPW // CH
RPT = NPAD // NS
DEGW = 16
ZB = 80

_mesh = plsc.VectorSubcoreMesh(
    core_axis_name="c", subcore_axis_name="s", num_cores=NC, num_subcores=NS)


SCH = 4
CHS = CH * SCH
NSUP = NCHUNK // SCH


def _agg_body(D, with_deg, g_h, src_h, dst_h, mp_h, *rest):
    if with_deg:
        (agg_out, deg_out, srcA, dstA, mpA, srcB, dstB, mpB,
         rows0, rows1, aggacc, semG0, semG1, semIA, semIB,
         ones, zdeg, degacc) = rest
    else:
        (agg_out, srcA, dstA, mpA, srcB, dstB, mpB,
         rows0, rows1, aggacc, semG0, semG1,
         semIA, semIB) = rest
    c = lax.axis_index("c")
    s = lax.axis_index("s")
    wid = c * NS + s
    ebase0 = wid * EPW
    rows = (rows0, rows1)
    semG = (semG0, semG1)
    bankA = (srcA, dstA, mpA, semIA)
    bankB = (srcB, dstB, mpB, semIB)

    def zrow(r, _):
        for k in range(D // 16):
            rows0[r, pl.ds(k * 16, 16)] = jnp.zeros((16,), jnp.float32)
        return 0
    lax.fori_loop(0, CH, zrow, 0)
    for k in range(RPT // CH):
        pltpu.sync_copy(rows0, aggacc.at[pl.ds(s * RPT + k * CH, CH)])
    if with_deg:
        def zd(i, _):
            zdeg[pl.ds(i * 16, 16)] = jnp.zeros((16,), jnp.float32)
            return 0
        lax.fori_loop(0, RPT // 16, zd, 0)
        pltpu.sync_copy(zdeg, degacc.at[pl.ds(s * RPT, RPT)])

        def od(i, _):
            ones[pl.ds(i * 16, 16)] = jnp.ones((16,), jnp.float32)
            return 0
        lax.fori_loop(0, CH // 16, od, 0)
    plsc.subcore_barrier()

    def loadidx(t_sup, bank):
        eb = ebase0 + t_sup * CHS
        pltpu.async_copy(src_h.at[pl.ds(eb, CHS)], bank[0], bank[3])
        pltpu.async_copy(dst_h.at[pl.ds(eb, CHS)], bank[1], bank[3])
        pltpu.async_copy(mp_h.at[pl.ds(eb, CHS)], bank[2], bank[3])

    def waitidx(bank):
        pltpu.make_async_copy(src_h.at[pl.ds(0, CHS)], bank[0], bank[3]).wait()
        pltpu.make_async_copy(dst_h.at[pl.ds(0, CHS)], bank[1], bank[3]).wait()
        pltpu.make_async_copy(mp_h.at[pl.ds(0, CHS)], bank[2], bank[3]).wait()

    def gstart(src_idx, b):
        pltpu.async_copy(g_h.at[src_idx], rows[b], semG[b])

    def work(src_idx, dst_idx, mpv, moff, b):
        pltpu.make_async_copy(g_h.at[src_idx], rows[b], semG[b]).wait()
        rb = rows[b]

        def erow(i, _):
            for u in range(8):
                e = i * 8 + u
                spl = plsc.load_gather(
                    mpv, [jnp.full((16,), moff + e, jnp.int32)])
                for k in range(D // 16):
                    sl = pl.ds(k * 16, 16)
                    rb[e, sl] = rb[e, sl] * spl
            return 0
        lax.fori_loop(0, CH // 8, erow, 0)
        pltpu.sync_copy(rb, aggacc.at[dst_idx], add=True)
        if with_deg:
            pltpu.sync_copy(ones, degacc.at[dst_idx], add=True)

    def do_super(cur, nxt, last=False):
        srcv, dstv, mpv = cur[:3]
        for j in range(SCH):
            b = j % 2
            ob = (j + 1) % 2
            if j < SCH - 1:
                gstart(srcv.at[pl.ds((j + 1) * CH, CH)], ob)
            elif not last:
                waitidx(nxt)
                gstart(nxt[0].at[pl.ds(0, CH)], ob)
            work(srcv.at[pl.ds(j * CH, CH)], dstv.at[pl.ds(j * CH, CH)],
                 mpv, j * CH, b)

    loadidx(0, bankA)
    waitidx(bankA)
    gstart(srcA.at[pl.ds(0, CH)], 0)

    def pairbody(p, _):
        loadidx(2 * p + 1, bankB)
        do_super(bankA, bankB)
        loadidx(2 * p + 2, bankA)
        do_super(bankB, bankA)
        return 0
    lax.fori_loop(0, (NSUP - 2) // 2, pairbody, 0)

    loadidx(NSUP - 1, bankB)
    do_super(bankA, bankB)
    do_super(bankB, bankA, last=True)
    plsc.subcore_barrier()

    for k in range(RPT // CH):
        start = s * RPT + k * CH
        pltpu.sync_copy(aggacc.at[pl.ds(start, CH)], rows0)
        pltpu.sync_copy(rows0, agg_out.at[pl.ds(c * NPAD + start, CH)])
    if with_deg:
        pltpu.sync_copy(degacc.at[pl.ds(s * RPT, RPT)], zdeg)
        pltpu.sync_copy(zdeg, deg_out.at[pl.ds(c * NPAD + s * RPT, RPT)])


def _make_agg(D, with_deg):
    out_type = [jax.ShapeDtypeStruct((NC * NPAD, D), jnp.float32)]
    if with_deg:
        out_type.append(jax.ShapeDtypeStruct((NC * NPAD,), jnp.float32))
    bank = [
        pltpu.VMEM((CHS,), jnp.int32),
        pltpu.VMEM((CHS,), jnp.int32),
        pltpu.VMEM((CHS,), jnp.float32),
    ]
    scratch = bank + bank + [
        pltpu.VMEM((CH, D), jnp.float32),
        pltpu.VMEM((CH, D), jnp.float32),
        pltpu.VMEM_SHARED((NPAD, D), jnp.float32),
    ] + [pltpu.SemaphoreType.DMA] * 4
    if with_deg:
        scratch += [
            pltpu.VMEM((CH,), jnp.float32),
            pltpu.VMEM((RPT,), jnp.float32),
            pltpu.VMEM_SHARED((NPAD,), jnp.float32),
        ]
    return pl.kernel(
        functools.partial(_agg_body, D, with_deg),
        out_type=tuple(out_type) if with_deg else out_type[0],
        mesh=_mesh,
        scratch_types=scratch,
        compiler_params=pltpu.CompilerParams(needs_layout_passes=False),
    )


_agg128d = _make_agg(128, True)
_agg128 = _make_agg(128, False)


def _maskmul_body(a, b, o):
    o[...] = a[...] * b[...]


def _maskmul(m1, m2):
    m1r = m1.reshape(E // 128, 128)
    m2r = m2.reshape(E // 128, 128)
    out = pl.pallas_call(
        _maskmul_body,
        grid=(1,),
        in_specs=[pl.BlockSpec((E // 128, 128), lambda i: (0, 0)),
                  pl.BlockSpec((E // 128, 128), lambda i: (0, 0))],
        out_specs=pl.BlockSpec((E // 128, 128), lambda i: (0, 0)),
        out_shape=jax.ShapeDtypeStruct((E // 128, 128), jnp.float32),
    )(m1r, m2r)
    return out.reshape(E)


def _matmul_body(x, w, o):
    o[...] = jnp.dot(x[...], w[...], preferred_element_type=jnp.float32)


def _matmul(x, w):
    m, k = x.shape
    n = w.shape[1]
    bm = 1024
    return pl.pallas_call(
        _matmul_body,
        grid=(m // bm,),
        in_specs=[pl.BlockSpec((bm, k), lambda i: (i, 0)),
                  pl.BlockSpec((k, n), lambda i: (0, 0))],
        out_specs=pl.BlockSpec((bm, n), lambda i: (i, 0)),
        out_shape=jax.ShapeDtypeStruct((m, n), jnp.float32),
    )(x, w)


def _fuse1_body(g0, agg, deg, eps, a, w, out):
    degv = deg[...]
    dsum = degv[0, :, 0:1] + degv[1, :, 0:1]
    inv = 1.0 / jnp.maximum(dsum, 1.0)
    aggv = agg[...]
    ag = (aggv[0] + aggv[1]) * inv
    pre = (1.0 + eps[0, 0]) * g0[...] + ag
    h0 = jnp.where(pre >= 0.0, pre, a[0, 0] * pre)
    out[...] = jnp.dot(h0, w[...], preferred_element_type=jnp.float32)


def _fuse1(g0, agg, deg, eps, a, wcat):
    bm = 1024
    return pl.pallas_call(
        _fuse1_body,
        grid=(NPAD // bm,),
        in_specs=[
            pl.BlockSpec((bm, 128), lambda i: (i, 0)),
            pl.BlockSpec((NC, bm, 128), lambda i: (0, i, 0)),
            pl.BlockSpec((NC, bm, 1), lambda i: (0, i, 0)),
            pl.BlockSpec((1, 1), lambda i: (0, 0), memory_space=pltpu.SMEM),
            pl.BlockSpec((1, 1), lambda i: (0, 0), memory_space=pltpu.SMEM),
            pl.BlockSpec((128, 128), lambda i: (0, 0)),
        ],
        out_specs=pl.BlockSpec((bm, 128), lambda i: (i, 0)),
        out_shape=jax.ShapeDtypeStruct((NPAD, 128), jnp.float32),
    )(g0, agg, deg, eps, a, wcat)


def _fuse2_body(g1p, agg, deg, eps, a, out):
    degv = deg[...]
    dsum = degv[0, :, 0:1] + degv[1, :, 0:1]
    inv = 1.0 / jnp.maximum(dsum, 1.0)
    y = g1p[...]
    aggv = agg[...]
    ag = (aggv[0] + aggv[1])[:, :64] * inv
    pre = (1.0 + eps[0, 0]) * y[:, :64] + ag
    h1 = jnp.where(pre >= 0.0, pre, a[0, 0] * pre)
    out[...] = (y[:, 64:] + h1) * 0.5


def _fuse2(g1p, agg, deg, eps, a):
    bm = 1024
    return pl.pallas_call(
        _fuse2_body,
        grid=(NPAD // bm,),
        in_specs=[
            pl.BlockSpec((bm, 128), lambda i: (i, 0)),
            pl.BlockSpec((NC, bm, 128), lambda i: (0, i, 0)),
            pl.BlockSpec((NC, bm, 1), lambda i: (0, i, 0)),
            pl.BlockSpec((1, 1), lambda i: (0, 0), memory_space=pltpu.SMEM),
            pl.BlockSpec((1, 1), lambda i: (0, 0), memory_space=pltpu.SMEM),
        ],
        out_specs=pl.BlockSpec((bm, 64), lambda i: (i, 0)),
        out_shape=jax.ShapeDtypeStruct((NPAD, 64), jnp.float32),
    )(g1p, agg, deg, eps, a)


def kernel(h, snorm_n, snorm_e, mask1, mask2, eps0, eps1, a0, a1,
           W0, W1, Wpred, edge_index):
    npadE = E2 - E
    src2 = jnp.concatenate([edge_index[0],
                            jnp.zeros((npadE,), jnp.int32)])
    dst2 = jnp.concatenate([edge_index[1],
                            jnp.full((npadE,), NPAD - 1, jnp.int32)])
    mp = jnp.concatenate([_maskmul(mask1.reshape(E), mask2.reshape(E)),
                          jnp.zeros((npadE,), jnp.float32)])
    hpad = jnp.pad(h, ((0, NPAD - N), (0, 0)))

    g0 = _matmul(hpad, W0)
    agg0f, degf = _agg128d(g0, src2, dst2, mp)
    agg0 = agg0f.reshape(NC, NPAD, 128)
    deg = degf.reshape(NC, NPAD, 1)

    wcat = jnp.concatenate([W1, Wpred], axis=1)
    g1p = _fuse1(g0, agg0, deg, eps0.reshape(1, 1), a0.reshape(1, 1), wcat)

    agg1f = _agg128(g1p, src2, dst2, mp)
    agg1 = agg1f.reshape(NC, NPAD, 128)

    score = _fuse2(g1p, agg1, deg, eps1.reshape(1, 1), a1.reshape(1, 1))
    return score[:N][None]
